# Initial kernel scaffold; baseline (speedup 1.0000x reference)
import jax
import jax.numpy as jnp
from jax.experimental import pallas as pl


def kernel(x, shuffle_idx):
    raise NotImplementedError("write your pallas kernel here")


# speedup vs baseline: 4.4075x; 4.4075x over previous
import functools

import jax
import jax.numpy as jnp
from jax import lax
from jax.experimental import pallas as pl
from jax.experimental.pallas import tpu as pltpu
from jax.experimental.pallas import tpu_sc as plsc

_NUM_CORES = 2
_NUM_SUBCORES = 16
_NUM_WORKERS = _NUM_CORES * _NUM_SUBCORES


def _sc_permute(flat, idx):
    c, n = flat.shape
    bpw = n // _NUM_WORKERS
    mesh = plsc.VectorSubcoreMesh(core_axis_name="core", subcore_axis_name="subcore")

    @functools.partial(
        pl.kernel,
        mesh=mesh,
        out_type=jax.ShapeDtypeStruct((c * n,), flat.dtype),
        scratch_types=[
            pltpu.VMEM((bpw,), jnp.int32),
            pltpu.VMEM((bpw,), jnp.float32),
            pltpu.SemaphoreType.DMA,
        ],
    )
    def k(x_hbm, idx_hbm, out_hbm, idx_v, vals_v, sem):
        wid = lax.axis_index("subcore") * _NUM_CORES + lax.axis_index("core")
        base = wid * bpw
        pltpu.sync_copy(idx_hbm.at[pl.ds(base, bpw)], idx_v)
        for ch in range(c):
            pltpu.async_copy(x_hbm.at[pl.ds(ch * n, n)].at[idx_v], vals_v, sem).wait()
            pltpu.sync_copy(vals_v, out_hbm.at[pl.ds(ch * n + base, bpw)])

    return k(flat.reshape(c * n), idx).reshape(c, n)


def kernel(x, shuffle_idx):
    c, w, h = x.shape
    flat = x.reshape(c, w * h)
    idx = shuffle_idx.astype(jnp.int32)
    out = _sc_permute(flat, idx)
    return out.reshape(c, w, h)


# speedup vs baseline: 4.4817x; 1.0168x over previous
import functools

import jax
import jax.numpy as jnp
from jax import lax
from jax.experimental import pallas as pl
from jax.experimental.pallas import tpu as pltpu
from jax.experimental.pallas import tpu_sc as plsc

_NUM_CORES = 2
_NUM_SUBCORES = 16
_NUM_WORKERS = _NUM_CORES * _NUM_SUBCORES


def _sc_permute(flat, idx):
    c, n = flat.shape
    bpw = n // _NUM_WORKERS
    mesh = plsc.VectorSubcoreMesh(core_axis_name="core", subcore_axis_name="subcore")

    @functools.partial(
        pl.kernel,
        mesh=mesh,
        out_type=jax.ShapeDtypeStruct((c * n,), flat.dtype),
        scratch_types=[
            pltpu.VMEM((bpw,), jnp.int32),
        ]
        + [pltpu.VMEM((bpw,), jnp.float32) for _ in range(c)]
        + [pltpu.SemaphoreType.DMA for _ in range(c)]
        + [pltpu.SemaphoreType.DMA],
    )
    def k(x_hbm, idx_hbm, out_hbm, idx_v, *rest):
        vals = rest[:c]
        gsems = rest[c : 2 * c]
        wsem = rest[2 * c]
        wid = lax.axis_index("subcore") * _NUM_CORES + lax.axis_index("core")
        base = wid * bpw
        pltpu.sync_copy(idx_hbm.at[pl.ds(base, bpw)], idx_v)
        gathers = [
            pltpu.async_copy(x_hbm.at[pl.ds(ch * n, n)].at[idx_v], vals[ch], gsems[ch])
            for ch in range(c)
        ]
        writes = []
        for ch in range(c):
            gathers[ch].wait()
            writes.append(
                pltpu.async_copy(vals[ch], out_hbm.at[pl.ds(ch * n + base, bpw)], wsem)
            )
        for wr in writes:
            wr.wait()

    return k(flat.reshape(c * n), idx).reshape(c, n)


def kernel(x, shuffle_idx):
    c, w, h = x.shape
    flat = x.reshape(c, w * h)
    idx = shuffle_idx.astype(jnp.int32)
    out = _sc_permute(flat, idx)
    return out.reshape(c, w, h)


# speedup vs baseline: 6.7724x; 1.5111x over previous
import functools

import jax
import jax.numpy as jnp
from jax import lax
from jax.experimental import pallas as pl
from jax.experimental.pallas import tpu as pltpu
from jax.experimental.pallas import tpu_sc as plsc

_NUM_CORES = 2
_NUM_SUBCORES = 16
_NUM_WORKERS = _NUM_CORES * _NUM_SUBCORES


def _sc_permute(flat, idx):
    c, n = flat.shape
    bpw = n // _NUM_WORKERS
    mesh = plsc.VectorSubcoreMesh(core_axis_name="core", subcore_axis_name="subcore")

    @functools.partial(
        pl.kernel,
        mesh=mesh,
        out_type=jax.ShapeDtypeStruct((c * n,), flat.dtype),
        scratch_types=[
            pltpu.VMEM((bpw,), jnp.int32),
        ]
        + [pltpu.VMEM((bpw,), jnp.float32) for _ in range(c)]
        + [pltpu.SemaphoreType.DMA for _ in range(c)]
        + [pltpu.SemaphoreType.DMA, pltpu.SemaphoreType.DMA]
        + [pltpu.VMEM_SHARED((c * n,), jnp.float32)],
    )
    def k(x_hbm, idx_hbm, out_hbm, idx_v, *rest):
        vals = rest[:c]
        gsems = rest[c : 2 * c]
        wsem = rest[2 * c]
        ssem = rest[2 * c + 1]
        x_sp = rest[2 * c + 2]
        sid = lax.axis_index("subcore")
        wid = sid * _NUM_CORES + lax.axis_index("core")
        base = wid * bpw
        share = (c * n) // _NUM_SUBCORES
        stage = pltpu.async_copy(
            x_hbm.at[pl.ds(sid * share, share)], x_sp.at[pl.ds(sid * share, share)], ssem
        )
        pltpu.sync_copy(idx_hbm.at[pl.ds(base, bpw)], idx_v)
        stage.wait()
        plsc.subcore_barrier()
        gathers = [
            pltpu.async_copy(x_sp.at[pl.ds(ch * n, n)].at[idx_v], vals[ch], gsems[ch])
            for ch in range(c)
        ]
        writes = []
        for ch in range(c):
            gathers[ch].wait()
            writes.append(
                pltpu.async_copy(vals[ch], out_hbm.at[pl.ds(ch * n + base, bpw)], wsem)
            )
        for wr in writes:
            wr.wait()

    return k(flat.reshape(c * n), idx).reshape(c, n)


def kernel(x, shuffle_idx):
    c, w, h = x.shape
    flat = x.reshape(c, w * h)
    idx = shuffle_idx.astype(jnp.int32)
    out = _sc_permute(flat, idx)
    return out.reshape(c, w, h)
